# baseline (speedup 1.0000x reference)
import functools

import jax
import jax.numpy as jnp
from jax import lax
from jax.experimental import pallas as pl
from jax.experimental.pallas import tpu as pltpu
from jax.experimental.pallas import tpu_sc as plsc

H = 128
BOND = 64
N_NODES = 10000
E = 320000
L = 16
HA = H + L

NC = 2
NS = 16
E_PER_SC = E // NC
E_PER_TILE = E_PER_SC // NS
B = 40
NBLK = E_PER_TILE // B
PAIRS = NBLK // 2
RPT = N_NODES // NS

_mesh = plsc.VectorSubcoreMesh(core_axis_name="c", subcore_axis_name="s")


@functools.partial(
    pl.kernel,
    out_type=(jax.ShapeDtypeStruct((NC, N_NODES, H), jnp.float32),
              jax.ShapeDtypeStruct((NC, N_NODES, L), jnp.float32)),
    mesh=_mesh,
    compiler_params=pltpu.CompilerParams(use_tc_tiling_on_sc=False,
                                         needs_layout_passes=False),
    scratch_types=[
        pltpu.VMEM((2, B), jnp.int32),
        pltpu.VMEM((2, B), jnp.int32),
        pltpu.VMEM((2, B, H), jnp.float32),
        pltpu.VMEM((2, B, H), jnp.float32),
        pltpu.VMEM((2, B, HA), jnp.float32),
        pltpu.VMEM((B, H), jnp.float32),
        pltpu.VMEM((B, L), jnp.float32),
        pltpu.VMEM((2, B), jnp.int32),
        pltpu.VMEM_SHARED((N_NODES, H), jnp.float32),
        pltpu.VMEM_SHARED((N_NODES, L), jnp.float32),
        pltpu.SemaphoreType.DMA,
        pltpu.SemaphoreType.DMA,
        pltpu.SemaphoreType.DMA,
        pltpu.SemaphoreType.DMA,
        pltpu.SemaphoreType.DMA,
    ],
)
def _sc_edges(pa, pb, bw_mat, row, col, out1, out2,
              idxr, idxc, abuf, bbuf, wbuf, ubuf, gubuf, sidx, S1, S2,
              semi0, semi1, semg0, semg1, sems):
    cid = lax.axis_index("c")
    sid = lax.axis_index("s")
    semi = (semi0, semi1)
    semg = (semg0, semg1)

    zv = jnp.zeros((L,), jnp.float32)

    def _zrow(i, carry):
        for j in range(H // L):
            ubuf[i, pl.ds(j * L, L)] = zv
        gubuf[i, pl.ds(0, L)] = zv
        return carry

    lax.fori_loop(0, B, _zrow, 0)
    nfull = RPT // B
    tail = RPT - nfull * B
    for j in range(nfull):
        pltpu.sync_copy(ubuf, S1.at[pl.ds(sid * RPT + j * B, B)])
        pltpu.sync_copy(gubuf, S2.at[pl.ds(sid * RPT + j * B, B)])
    pltpu.sync_copy(ubuf.at[pl.ds(0, tail)],
                    S1.at[pl.ds(sid * RPT + nfull * B, tail)])
    pltpu.sync_copy(gubuf.at[pl.ds(0, tail)],
                    S2.at[pl.ds(sid * RPT + nfull * B, tail)])
    plsc.subcore_barrier()

    ebase = (cid * NS + sid) * E_PER_TILE

    def _issue_idx(blk, b):
        off = ebase + blk * B
        pltpu.async_copy(row.at[pl.ds(off, B)], idxr.at[b], semi[b])
        pltpu.async_copy(col.at[pl.ds(off, B)], idxc.at[b], semi[b])

    def _wait_idx(b):
        pltpu.make_async_copy(row.at[pl.ds(0, B)], idxr.at[b], semi[b]).wait()
        pltpu.make_async_copy(col.at[pl.ds(0, B)], idxc.at[b], semi[b]).wait()

    def _issue_gath(blk, b):
        off = ebase + blk * B
        pltpu.async_copy(pa.at[idxr.at[b]], abuf.at[b], semg[b])
        pltpu.async_copy(pb.at[idxc.at[b]], bbuf.at[b], semg[b])
        pltpu.async_copy(bw_mat.at[pl.ds(off, B)], wbuf.at[b], semg[b])

    def _wait_gath(b):
        pltpu.make_async_copy(pa.at[idxr.at[b]], abuf.at[b], semg[b]).wait()
        pltpu.make_async_copy(pb.at[idxc.at[b]], bbuf.at[b], semg[b]).wait()
        pltpu.make_async_copy(bw_mat.at[pl.ds(0, B)], wbuf.at[b],
                              semg[b]).wait()

    def _issue_scat(b):
        pltpu.async_copy(ubuf, S1.at[sidx.at[b]], sems, add=True)
        pltpu.async_copy(gubuf, S2.at[sidx.at[b]], sems, add=True)

    def _wait_scat(b):
        pltpu.make_async_copy(ubuf, S1.at[sidx.at[b]], sems).wait()
        pltpu.make_async_copy(gubuf, S2.at[sidx.at[b]], sems).wait()

    def _snap_idx(b):
        for o in range(0, B - L + 1, L):
            sidx[b, pl.ds(o, L)] = idxr[b, pl.ds(o, L)]
        if B % L:
            sidx[b, pl.ds(B - L, L)] = idxr[b, pl.ds(B - L, L)]

    _col_h = jnp.full((L,), H, dtype=jnp.int32)

    def _compute(b):
        def _edge(e, c2):
            ev = lax.broadcast_in_dim(e, (L,), ())
            zv = plsc.load_gather(wbuf.at[b], [ev, _col_h])
            gv = 1.0 / (1.0 + jnp.exp(-zv))
            for j in range(H // L):
                x = (abuf[b, e, pl.ds(j * L, L)] + bbuf[b, e, pl.ds(j * L, L)]
                     + wbuf[b, e, pl.ds(j * L, L)])
                ubuf[e, pl.ds(j * L, L)] = jnp.maximum(x, 0.0) * gv
            gubuf[e, pl.ds(0, L)] = gv
            return c2

        lax.fori_loop(0, B, _edge, 0)

    def _half(j, b, wait_scat_prev, prefetch):
        _wait_gath(b)
        _snap_idx(b)
        if prefetch:
            _issue_idx(j + 2, b)
        if wait_scat_prev:
            _wait_scat(b)
        _compute(b)
        if prefetch:
            _wait_idx(b)
            _issue_gath(j + 2, b)
        _issue_scat(b)

    _issue_idx(0, 0)
    _issue_idx(1, 1)
    _wait_idx(0)
    _issue_gath(0, 0)
    _wait_idx(1)
    _issue_gath(1, 1)

    _half(0, 0, False, True)
    _half(1, 1, True, True)

    def _pair(k, carry):
        j = 2 * k
        _half(j, 0, True, True)
        _half(j + 1, 1, True, True)
        return carry

    lax.fori_loop(1, PAIRS - 1, _pair, 0)

    _half(NBLK - 2, 0, True, False)
    _half(NBLK - 1, 1, True, False)
    _wait_scat(1)

    plsc.subcore_barrier()
    pltpu.sync_copy(S1.at[pl.ds(sid * RPT, RPT)],
                    out1.at[cid, pl.ds(sid * RPT, RPT)])
    pltpu.sync_copy(S2.at[pl.ds(sid * RPT, RPT)],
                    out2.at[cid, pl.ds(sid * RPT, RPT)])


NB = 1000
EB = 1280


def _node_proj_body(h_ref, wa_ref, wb_ref, oa_ref, ob_ref):
    hh = h_ref[...]
    oa_ref[...] = jnp.dot(hh, wa_ref[...], preferred_element_type=jnp.float32)
    ob_ref[...] = jnp.dot(hh, wb_ref[...], preferred_element_type=jnp.float32)


def _node_proj(h, W1a, W1b):
    return pl.pallas_call(
        _node_proj_body,
        grid=(N_NODES // NB,),
        in_specs=[pl.BlockSpec((NB, H), lambda i: (i, 0)),
                  pl.BlockSpec((H, H), lambda i: (0, 0)),
                  pl.BlockSpec((H, H), lambda i: (0, 0))],
        out_specs=[pl.BlockSpec((NB, H), lambda i: (i, 0)),
                   pl.BlockSpec((NB, H), lambda i: (i, 0))],
        out_shape=[jax.ShapeDtypeStruct((N_NODES, H), jnp.float32),
                   jax.ShapeDtypeStruct((N_NODES, H), jnp.float32)],
    )(h, W1a, W1b)


def _edge_proj_body(bond_ref, w_ref, b_ref, ow_ref):
    ow_ref[...] = (jnp.dot(bond_ref[...], w_ref[...],
                           preferred_element_type=jnp.float32) + b_ref[...])


def _edge_proj(bond_emb, W1c, b1, Wa, ba):
    w_ext = jnp.concatenate(
        [W1c, Wa, jnp.zeros((BOND, HA - H - 1), jnp.float32)], axis=1)
    b_ext = jnp.concatenate(
        [b1, ba, jnp.zeros((HA - H - 1,), jnp.float32)], axis=0)
    return pl.pallas_call(
        _edge_proj_body,
        grid=(E // EB,),
        in_specs=[pl.BlockSpec((EB, BOND), lambda i: (i, 0)),
                  pl.BlockSpec((BOND, HA), lambda i: (0, 0)),
                  pl.BlockSpec((1, HA), lambda i: (0, 0))],
        out_specs=pl.BlockSpec((EB, HA), lambda i: (i, 0)),
        out_shape=jax.ShapeDtypeStruct((E, HA), jnp.float32),
    )(bond_emb, w_ext, b_ext.reshape(1, HA))


def _epilogue_body(h_ref, s1_ref, s2_ref, w2_ref, b2_ref, o_ref):
    t = s1_ref[0] + s1_ref[1]
    cnt = (s2_ref[0] + s2_ref[1])[:, :1]
    o_ref[...] = (h_ref[...]
                  + jnp.dot(t, w2_ref[...], preferred_element_type=jnp.float32)
                  + cnt * b2_ref[...])


def _epilogue(h, s1_parts, s2_parts, W2, b2):
    return pl.pallas_call(
        _epilogue_body,
        grid=(N_NODES // NB,),
        in_specs=[pl.BlockSpec((NB, H), lambda i: (i, 0)),
                  pl.BlockSpec((NC, NB, H), lambda i: (0, i, 0)),
                  pl.BlockSpec((NC, NB, L), lambda i: (0, i, 0)),
                  pl.BlockSpec((H, H), lambda i: (0, 0)),
                  pl.BlockSpec((1, H), lambda i: (0, 0))],
        out_specs=pl.BlockSpec((NB, H), lambda i: (i, 0)),
        out_shape=jax.ShapeDtypeStruct((N_NODES, H), jnp.float32),
    )(h, s1_parts, s2_parts, W2, b2.reshape(1, H))


def kernel(h, edge_index, bond_emb, W1, b1, W2, b2, Wa, ba):
    row = edge_index[0].astype(jnp.int32)
    col = edge_index[1].astype(jnp.int32)
    W1a = W1[:H]
    W1b = W1[H:2 * H]
    W1c = W1[2 * H:]
    pa, pb = _node_proj(h, W1a, W1b)
    bondw = _edge_proj(bond_emb, W1c, b1, Wa, ba)
    s1_parts, s2_parts = _sc_edges(pa, pb, bondw, row, col)
    return _epilogue(h, s1_parts, s2_parts, W2, b2)


# speedup vs baseline: 1.4181x; 1.4181x over previous
import functools

import jax
import jax.numpy as jnp
from jax import lax
from jax.experimental import pallas as pl
from jax.experimental.pallas import tpu as pltpu
from jax.experimental.pallas import tpu_sc as plsc

H = 128
BOND = 64
N_NODES = 10000
E = 320000
L = 16
HA = H + L

NC = 2
NS = 16
E_PER_SC = E // NC
E_PER_TILE = E_PER_SC // NS
B = 40
NBLK = E_PER_TILE // B
PAIRS = NBLK // 2
RPT = N_NODES // NS

_mesh = plsc.VectorSubcoreMesh(core_axis_name="c", subcore_axis_name="s")


@functools.partial(
    pl.kernel,
    out_type=(jax.ShapeDtypeStruct((NC, N_NODES, H), jnp.float32),
              jax.ShapeDtypeStruct((NC, N_NODES, L), jnp.float32)),
    mesh=_mesh,
    compiler_params=pltpu.CompilerParams(use_tc_tiling_on_sc=False,
                                         needs_layout_passes=False),
    scratch_types=[
        pltpu.VMEM((2, B), jnp.int32),
        pltpu.VMEM((2, B), jnp.int32),
        pltpu.VMEM((2, B, H), jnp.float32),
        pltpu.VMEM((2, B, H), jnp.float32),
        pltpu.VMEM((2, B, H), jnp.float32),
        pltpu.VMEM((2, B), jnp.float32),
        pltpu.VMEM((B, H), jnp.float32),
        pltpu.VMEM((B, L), jnp.float32),
        pltpu.VMEM((2, B), jnp.int32),
        pltpu.VMEM_SHARED((N_NODES, H), jnp.float32),
        pltpu.VMEM_SHARED((N_NODES, L), jnp.float32),
        pltpu.SemaphoreType.DMA,
        pltpu.SemaphoreType.DMA,
        pltpu.SemaphoreType.DMA,
        pltpu.SemaphoreType.DMA,
        pltpu.SemaphoreType.DMA,
    ],
)
def _sc_edges(pa, pb, bw_mat, gate, row, col, out1, out2,
              idxr, idxc, abuf, bbuf, wbuf, gbuf, ubuf, gubuf, sidx, S1, S2,
              semi0, semi1, semg0, semg1, sems):
    cid = lax.axis_index("c")
    sid = lax.axis_index("s")
    semi = (semi0, semi1)
    semg = (semg0, semg1)

    zv = jnp.zeros((L,), jnp.float32)

    def _zrow(i, carry):
        for j in range(H // L):
            ubuf[i, pl.ds(j * L, L)] = zv
        gubuf[i, pl.ds(0, L)] = zv
        return carry

    lax.fori_loop(0, B, _zrow, 0)
    nfull = RPT // B
    tail = RPT - nfull * B
    for j in range(nfull):
        pltpu.sync_copy(ubuf, S1.at[pl.ds(sid * RPT + j * B, B)])
        pltpu.sync_copy(gubuf, S2.at[pl.ds(sid * RPT + j * B, B)])
    pltpu.sync_copy(ubuf.at[pl.ds(0, tail)],
                    S1.at[pl.ds(sid * RPT + nfull * B, tail)])
    pltpu.sync_copy(gubuf.at[pl.ds(0, tail)],
                    S2.at[pl.ds(sid * RPT + nfull * B, tail)])
    plsc.subcore_barrier()

    ebase = (cid * NS + sid) * E_PER_TILE

    def _issue_idx(blk, b):
        off = ebase + blk * B
        pltpu.async_copy(row.at[pl.ds(off, B)], idxr.at[b], semi[b])
        pltpu.async_copy(col.at[pl.ds(off, B)], idxc.at[b], semi[b])

    def _wait_idx(b):
        pltpu.make_async_copy(row.at[pl.ds(0, B)], idxr.at[b], semi[b]).wait()
        pltpu.make_async_copy(col.at[pl.ds(0, B)], idxc.at[b], semi[b]).wait()

    def _issue_gath(blk, b):
        off = ebase + blk * B
        pltpu.async_copy(pa.at[idxr.at[b]], abuf.at[b], semg[b])
        pltpu.async_copy(pb.at[idxc.at[b]], bbuf.at[b], semg[b])
        pltpu.async_copy(bw_mat.at[pl.ds(off, B)], wbuf.at[b], semg[b])
        pltpu.async_copy(gate.at[pl.ds(off, B)], gbuf.at[b], semg[b])

    def _wait_gath(b):
        pltpu.make_async_copy(pa.at[idxr.at[b]], abuf.at[b], semg[b]).wait()
        pltpu.make_async_copy(pb.at[idxc.at[b]], bbuf.at[b], semg[b]).wait()
        pltpu.make_async_copy(bw_mat.at[pl.ds(0, B)], wbuf.at[b],
                              semg[b]).wait()
        pltpu.make_async_copy(gate.at[pl.ds(0, B)], gbuf.at[b], semg[b]).wait()

    def _issue_scat(b):
        pltpu.async_copy(ubuf, S1.at[sidx.at[b]], sems, add=True)
        pltpu.async_copy(gubuf, S2.at[sidx.at[b]], sems, add=True)

    def _wait_scat(b):
        pltpu.make_async_copy(ubuf, S1.at[sidx.at[b]], sems).wait()
        pltpu.make_async_copy(gubuf, S2.at[sidx.at[b]], sems).wait()

    def _snap_idx(b):
        for o in range(0, B - L + 1, L):
            sidx[b, pl.ds(o, L)] = idxr[b, pl.ds(o, L)]
        if B % L:
            sidx[b, pl.ds(B - L, L)] = idxr[b, pl.ds(B - L, L)]

    def _compute(b):
        def _edge(e, c2):
            gv = plsc.load_gather(gbuf.at[b],
                                  [lax.broadcast_in_dim(e, (L,), ())])
            for j in range(H // L):
                x = (abuf[b, e, pl.ds(j * L, L)] + bbuf[b, e, pl.ds(j * L, L)]
                     + wbuf[b, e, pl.ds(j * L, L)])
                ubuf[e, pl.ds(j * L, L)] = jnp.maximum(x, 0.0) * gv
            gubuf[e, pl.ds(0, L)] = gv
            return c2

        lax.fori_loop(0, B, _edge, 0)

    def _half(j, b, wait_scat_prev, prefetch):
        _wait_gath(b)
        _snap_idx(b)
        if prefetch:
            _issue_idx(j + 2, b)
        if wait_scat_prev:
            _wait_scat(b)
        _compute(b)
        if prefetch:
            _wait_idx(b)
            _issue_gath(j + 2, b)
        _issue_scat(b)

    _issue_idx(0, 0)
    _issue_idx(1, 1)
    _wait_idx(0)
    _issue_gath(0, 0)
    _wait_idx(1)
    _issue_gath(1, 1)

    _half(0, 0, False, True)
    _half(1, 1, True, True)

    def _pair(k, carry):
        j = 2 * k
        _half(j, 0, True, True)
        _half(j + 1, 1, True, True)
        return carry

    lax.fori_loop(1, PAIRS - 1, _pair, 0)

    _half(NBLK - 2, 0, True, False)
    _half(NBLK - 1, 1, True, False)
    _wait_scat(1)

    plsc.subcore_barrier()
    pltpu.sync_copy(S1.at[pl.ds(sid * RPT, RPT)],
                    out1.at[cid, pl.ds(sid * RPT, RPT)])
    pltpu.sync_copy(S2.at[pl.ds(sid * RPT, RPT)],
                    out2.at[cid, pl.ds(sid * RPT, RPT)])


NB = 1000
EB = 3200


def _node_proj_body(h_ref, wa_ref, wb_ref, oa_ref, ob_ref):
    hh = h_ref[...]
    oa_ref[...] = jnp.dot(hh, wa_ref[...], preferred_element_type=jnp.float32)
    ob_ref[...] = jnp.dot(hh, wb_ref[...], preferred_element_type=jnp.float32)


def _node_proj(h, W1a, W1b):
    return pl.pallas_call(
        _node_proj_body,
        grid=(N_NODES // NB,),
        in_specs=[pl.BlockSpec((NB, H), lambda i: (i, 0)),
                  pl.BlockSpec((H, H), lambda i: (0, 0)),
                  pl.BlockSpec((H, H), lambda i: (0, 0))],
        out_specs=[pl.BlockSpec((NB, H), lambda i: (i, 0)),
                   pl.BlockSpec((NB, H), lambda i: (i, 0))],
        out_shape=[jax.ShapeDtypeStruct((N_NODES, H), jnp.float32),
                   jax.ShapeDtypeStruct((N_NODES, H), jnp.float32)],
    )(h, W1a, W1b)


def _edge_proj_body(bond_ref, w1c_ref, b1_ref, wa_ref, ba_ref, ow_ref, og_ref):
    bond = bond_ref[...]
    ow_ref[...] = (jnp.dot(bond, w1c_ref[...],
                           preferred_element_type=jnp.float32) + b1_ref[...])
    z = (jnp.dot(bond, wa_ref[...], preferred_element_type=jnp.float32)
         + ba_ref[...])
    og_ref[...] = jax.nn.sigmoid(z)


def _edge_proj(bond_emb, W1c, b1, Wa, ba):
    return pl.pallas_call(
        _edge_proj_body,
        grid=(E // EB,),
        in_specs=[pl.BlockSpec((EB, BOND), lambda i: (i, 0)),
                  pl.BlockSpec((BOND, H), lambda i: (0, 0)),
                  pl.BlockSpec((1, H), lambda i: (0, 0)),
                  pl.BlockSpec((BOND, 1), lambda i: (0, 0)),
                  pl.BlockSpec((1, 1), lambda i: (0, 0))],
        out_specs=[pl.BlockSpec((EB, H), lambda i: (i, 0)),
                   pl.BlockSpec((EB, 1), lambda i: (i, 0))],
        out_shape=[jax.ShapeDtypeStruct((E, H), jnp.float32),
                   jax.ShapeDtypeStruct((E, 1), jnp.float32)],
    )(bond_emb, W1c, b1.reshape(1, H), Wa, ba.reshape(1, 1))


def _epilogue_body(h_ref, s1_ref, s2_ref, w2_ref, b2_ref, o_ref):
    t = s1_ref[0] + s1_ref[1]
    cnt = (s2_ref[0] + s2_ref[1])[:, :1]
    o_ref[...] = (h_ref[...]
                  + jnp.dot(t, w2_ref[...], preferred_element_type=jnp.float32)
                  + cnt * b2_ref[...])


def _epilogue(h, s1_parts, s2_parts, W2, b2):
    return pl.pallas_call(
        _epilogue_body,
        grid=(N_NODES // NB,),
        in_specs=[pl.BlockSpec((NB, H), lambda i: (i, 0)),
                  pl.BlockSpec((NC, NB, H), lambda i: (0, i, 0)),
                  pl.BlockSpec((NC, NB, L), lambda i: (0, i, 0)),
                  pl.BlockSpec((H, H), lambda i: (0, 0)),
                  pl.BlockSpec((1, H), lambda i: (0, 0))],
        out_specs=pl.BlockSpec((NB, H), lambda i: (i, 0)),
        out_shape=jax.ShapeDtypeStruct((N_NODES, H), jnp.float32),
    )(h, s1_parts, s2_parts, W2, b2.reshape(1, H))


def kernel(h, edge_index, bond_emb, W1, b1, W2, b2, Wa, ba):
    row = edge_index[0].astype(jnp.int32)
    col = edge_index[1].astype(jnp.int32)
    W1a = W1[:H]
    W1b = W1[H:2 * H]
    W1c = W1[2 * H:]
    pa, pb = _node_proj(h, W1a, W1b)
    bondw, gate2d = _edge_proj(bond_emb, W1c, b1, Wa, ba)
    gate = gate2d.reshape(E)
    s1_parts, s2_parts = _sc_edges(pa, pb, bondw, gate, row, col)
    return _epilogue(h, s1_parts, s2_parts, W2, b2)


# speedup vs baseline: 1.6490x; 1.1628x over previous
import functools

import jax
import jax.numpy as jnp
from jax import lax
from jax.experimental import pallas as pl
from jax.experimental.pallas import tpu as pltpu
from jax.experimental.pallas import tpu_sc as plsc

H = 128
BOND = 64
N_NODES = 10000
E = 320000
L = 16
HA = H + L

NC = 2
NS = 16
E_PER_SC = E // NC
E_PER_TILE = E_PER_SC // NS
B = 80
NBLK = E_PER_TILE // B
RPT = N_NODES // NS

_mesh = plsc.VectorSubcoreMesh(core_axis_name="c", subcore_axis_name="s")


@functools.partial(
    pl.kernel,
    out_type=(jax.ShapeDtypeStruct((NC, N_NODES, H), jnp.float32),
              jax.ShapeDtypeStruct((NC, N_NODES, L), jnp.float32)),
    mesh=_mesh,
    compiler_params=pltpu.CompilerParams(use_tc_tiling_on_sc=False,
                                         needs_layout_passes=False),
    scratch_types=[
        pltpu.VMEM((2, B), jnp.int32),
        pltpu.VMEM((2, B), jnp.int32),
        pltpu.VMEM((2, B, H), jnp.float32),
        pltpu.VMEM((2, B), jnp.float32),
        pltpu.VMEM((B, H), jnp.float32),
        pltpu.VMEM((B, L), jnp.float32),
        pltpu.VMEM((2, B), jnp.int32),
        pltpu.VMEM_SHARED((N_NODES, H), jnp.float32),
        pltpu.VMEM_SHARED((N_NODES, L), jnp.float32),
        pltpu.SemaphoreType.DMA,
        pltpu.SemaphoreType.DMA,
        pltpu.SemaphoreType.DMA,
        pltpu.SemaphoreType.DMA,
        pltpu.SemaphoreType.DMA,
        pltpu.SemaphoreType.DMA,
        pltpu.SemaphoreType.DMA,
    ],
)
def _sc_edges(pa, pb, bw_mat, gate, row, col, out1, out2,
              idxr, idxc, wbuf, gbuf, ubuf, gubuf, sidx, S1, S2,
              semi0, semi1, semw0, semw1, semg0, semg1, sems):
    cid = lax.axis_index("c")
    sid = lax.axis_index("s")
    semi = (semi0, semi1)
    semw = (semw0, semw1)
    semg = (semg0, semg1)

    zv = jnp.zeros((L,), jnp.float32)

    def _zrow(i, carry):
        for j in range(H // L):
            ubuf[i, pl.ds(j * L, L)] = zv
        gubuf[i, pl.ds(0, L)] = zv
        return carry

    lax.fori_loop(0, B, _zrow, 0)
    nfull = RPT // B
    tail = RPT - nfull * B
    for j in range(nfull):
        pltpu.sync_copy(ubuf, S1.at[pl.ds(sid * RPT + j * B, B)])
        pltpu.sync_copy(gubuf, S2.at[pl.ds(sid * RPT + j * B, B)])
    pltpu.sync_copy(ubuf.at[pl.ds(0, tail)],
                    S1.at[pl.ds(sid * RPT + nfull * B, tail)])
    pltpu.sync_copy(gubuf.at[pl.ds(0, tail)],
                    S2.at[pl.ds(sid * RPT + nfull * B, tail)])
    plsc.subcore_barrier()

    ebase = (cid * NS + sid) * E_PER_TILE

    def _issue_idx(blk, b):
        off = ebase + blk * B
        pltpu.async_copy(row.at[pl.ds(off, B)], idxr.at[b], semi[b])
        pltpu.async_copy(col.at[pl.ds(off, B)], idxc.at[b], semi[b])

    def _wait_idx(b):
        pltpu.make_async_copy(row.at[pl.ds(0, B)], idxr.at[b], semi[b]).wait()
        pltpu.make_async_copy(col.at[pl.ds(0, B)], idxc.at[b], semi[b]).wait()

    def _issue_w(blk, b):
        off = ebase + blk * B
        pltpu.async_copy(bw_mat.at[pl.ds(off, B)], wbuf.at[b], semw[b])
        pltpu.async_copy(gate.at[pl.ds(off, B)], gbuf.at[b], semw[b])

    def _wait_w(b):
        pltpu.make_async_copy(bw_mat.at[pl.ds(0, B)], wbuf.at[b],
                              semw[b]).wait()
        pltpu.make_async_copy(gate.at[pl.ds(0, B)], gbuf.at[b], semw[b]).wait()

    def _issue_adds(b):
        pltpu.async_copy(pa.at[idxr.at[b]], wbuf.at[b], semg[b], add=True)
        pltpu.async_copy(pb.at[idxc.at[b]], wbuf.at[b], semg[b], add=True)

    def _wait_adds(b):
        pltpu.make_async_copy(pa.at[idxr.at[b]], wbuf.at[b], semg[b]).wait()
        pltpu.make_async_copy(pb.at[idxc.at[b]], wbuf.at[b], semg[b]).wait()

    def _issue_scat(b):
        pltpu.async_copy(ubuf, S1.at[sidx.at[b]], sems, add=True)
        pltpu.async_copy(gubuf, S2.at[sidx.at[b]], sems, add=True)

    def _wait_scat(b):
        pltpu.make_async_copy(ubuf, S1.at[sidx.at[b]], sems).wait()
        pltpu.make_async_copy(gubuf, S2.at[sidx.at[b]], sems).wait()

    def _snap_idx(b):
        for o in range(0, B - L + 1, L):
            sidx[b, pl.ds(o, L)] = idxr[b, pl.ds(o, L)]
        if B % L:
            sidx[b, pl.ds(B - L, L)] = idxr[b, pl.ds(B - L, L)]

    def _compute(b):
        def _edge(e, c2):
            gv = plsc.load_gather(gbuf.at[b],
                                  [lax.broadcast_in_dim(e, (L,), ())])
            for j in range(H // L):
                x = wbuf[b, e, pl.ds(j * L, L)]
                ubuf[e, pl.ds(j * L, L)] = jnp.maximum(x, 0.0) * gv
            gubuf[e, pl.ds(0, L)] = gv
            return c2

        lax.fori_loop(0, B, _edge, 0)

    def _half(j, b, wait_scat_prev, prefetch, adds_next):
        _wait_adds(b)
        _snap_idx(b)
        if prefetch:
            _issue_idx(j + 2, b)
        if adds_next:
            _wait_w(1 - b)
            _issue_adds(1 - b)
        if wait_scat_prev:
            _wait_scat(b)
        _compute(b)
        if prefetch:
            _wait_idx(b)
            _issue_w(j + 2, b)
        _issue_scat(b)

    _issue_idx(0, 0)
    _issue_idx(1, 1)
    _issue_w(0, 0)
    _issue_w(1, 1)
    _wait_idx(0)
    _wait_idx(1)
    _wait_w(0)
    _issue_adds(0)

    _half(0, 0, False, True, True)
    _half(1, 1, True, True, True)

    def _pair(k, carry):
        j = 2 * k
        _half(j, 0, True, True, True)
        _half(j + 1, 1, True, True, True)
        return carry

    lax.fori_loop(1, (NBLK - 3) // 2, _pair, 0)

    _half(NBLK - 3, 0, True, True, True)
    _half(NBLK - 2, 1, True, False, True)
    _half(NBLK - 1, 0, True, False, False)
    _wait_scat(0)

    plsc.subcore_barrier()
    pltpu.sync_copy(S1.at[pl.ds(sid * RPT, RPT)],
                    out1.at[cid, pl.ds(sid * RPT, RPT)])
    pltpu.sync_copy(S2.at[pl.ds(sid * RPT, RPT)],
                    out2.at[cid, pl.ds(sid * RPT, RPT)])


NB = 1000
EB = 3200


def _node_proj_body(h_ref, wa_ref, wb_ref, oa_ref, ob_ref):
    hh = h_ref[...]
    oa_ref[...] = jnp.dot(hh, wa_ref[...], preferred_element_type=jnp.float32)
    ob_ref[...] = jnp.dot(hh, wb_ref[...], preferred_element_type=jnp.float32)


def _node_proj(h, W1a, W1b):
    return pl.pallas_call(
        _node_proj_body,
        grid=(N_NODES // NB,),
        in_specs=[pl.BlockSpec((NB, H), lambda i: (i, 0)),
                  pl.BlockSpec((H, H), lambda i: (0, 0)),
                  pl.BlockSpec((H, H), lambda i: (0, 0))],
        out_specs=[pl.BlockSpec((NB, H), lambda i: (i, 0)),
                   pl.BlockSpec((NB, H), lambda i: (i, 0))],
        out_shape=[jax.ShapeDtypeStruct((N_NODES, H), jnp.float32),
                   jax.ShapeDtypeStruct((N_NODES, H), jnp.float32)],
    )(h, W1a, W1b)


def _edge_proj_body(bond_ref, w1c_ref, b1_ref, wa_ref, ba_ref, ow_ref, og_ref):
    bond = bond_ref[...]
    ow_ref[...] = (jnp.dot(bond, w1c_ref[...],
                           preferred_element_type=jnp.float32) + b1_ref[...])
    z = (jnp.dot(bond, wa_ref[...], preferred_element_type=jnp.float32)
         + ba_ref[...])
    og_ref[...] = jax.nn.sigmoid(z)


def _edge_proj(bond_emb, W1c, b1, Wa, ba):
    return pl.pallas_call(
        _edge_proj_body,
        grid=(E // EB,),
        in_specs=[pl.BlockSpec((EB, BOND), lambda i: (i, 0)),
                  pl.BlockSpec((BOND, H), lambda i: (0, 0)),
                  pl.BlockSpec((1, H), lambda i: (0, 0)),
                  pl.BlockSpec((BOND, 1), lambda i: (0, 0)),
                  pl.BlockSpec((1, 1), lambda i: (0, 0))],
        out_specs=[pl.BlockSpec((EB, H), lambda i: (i, 0)),
                   pl.BlockSpec((EB, 1), lambda i: (i, 0))],
        out_shape=[jax.ShapeDtypeStruct((E, H), jnp.float32),
                   jax.ShapeDtypeStruct((E, 1), jnp.float32)],
    )(bond_emb, W1c, b1.reshape(1, H), Wa, ba.reshape(1, 1))


def _epilogue_body(h_ref, s1_ref, s2_ref, w2_ref, b2_ref, o_ref):
    t = s1_ref[0] + s1_ref[1]
    cnt = (s2_ref[0] + s2_ref[1])[:, :1]
    o_ref[...] = (h_ref[...]
                  + jnp.dot(t, w2_ref[...], preferred_element_type=jnp.float32)
                  + cnt * b2_ref[...])


def _epilogue(h, s1_parts, s2_parts, W2, b2):
    return pl.pallas_call(
        _epilogue_body,
        grid=(N_NODES // NB,),
        in_specs=[pl.BlockSpec((NB, H), lambda i: (i, 0)),
                  pl.BlockSpec((NC, NB, H), lambda i: (0, i, 0)),
                  pl.BlockSpec((NC, NB, L), lambda i: (0, i, 0)),
                  pl.BlockSpec((H, H), lambda i: (0, 0)),
                  pl.BlockSpec((1, H), lambda i: (0, 0))],
        out_specs=pl.BlockSpec((NB, H), lambda i: (i, 0)),
        out_shape=jax.ShapeDtypeStruct((N_NODES, H), jnp.float32),
    )(h, s1_parts, s2_parts, W2, b2.reshape(1, H))


def kernel(h, edge_index, bond_emb, W1, b1, W2, b2, Wa, ba):
    row = edge_index[0].astype(jnp.int32)
    col = edge_index[1].astype(jnp.int32)
    W1a = W1[:H]
    W1b = W1[H:2 * H]
    W1c = W1[2 * H:]
    pa, pb = _node_proj(h, W1a, W1b)
    bondw, gate2d = _edge_proj(bond_emb, W1c, b1, Wa, ba)
    gate = gate2d.reshape(E)
    s1_parts, s2_parts = _sc_edges(pa, pb, bondw, gate, row, col)
    return _epilogue(h, s1_parts, s2_parts, W2, b2)


# speedup vs baseline: 1.7766x; 1.0774x over previous
import functools

import jax
import jax.numpy as jnp
from jax import lax
from jax.experimental import pallas as pl
from jax.experimental.pallas import tpu as pltpu
from jax.experimental.pallas import tpu_sc as plsc

H = 128
BOND = 64
N_NODES = 10000
E = 320000
L = 16
HA = H + L

NC = 2
NS = 16
E_PER_SC = E // NC
E_PER_TILE = E_PER_SC // NS
B = 80
NBLK = E_PER_TILE // B
RPT = N_NODES // NS

_mesh = plsc.VectorSubcoreMesh(core_axis_name="c", subcore_axis_name="s")


@functools.partial(
    pl.kernel,
    out_type=(jax.ShapeDtypeStruct((NC, N_NODES, H), jnp.float32),
              jax.ShapeDtypeStruct((NC, N_NODES, L), jnp.float32)),
    mesh=_mesh,
    compiler_params=pltpu.CompilerParams(use_tc_tiling_on_sc=False,
                                         needs_layout_passes=False),
    scratch_types=[
        pltpu.VMEM((2, B), jnp.int32),
        pltpu.VMEM((2, B), jnp.int32),
        pltpu.VMEM((2, B, H), jnp.float32),
        pltpu.VMEM((2, B), jnp.float32),
        pltpu.VMEM((B, H), jnp.float32),
        pltpu.VMEM((B, L), jnp.float32),
        pltpu.VMEM((2, B), jnp.int32),
        pltpu.VMEM_SHARED((N_NODES, H), jnp.float32),
        pltpu.VMEM_SHARED((N_NODES, L), jnp.float32),
        pltpu.SemaphoreType.DMA,
        pltpu.SemaphoreType.DMA,
        pltpu.SemaphoreType.DMA,
        pltpu.SemaphoreType.DMA,
        pltpu.SemaphoreType.DMA,
        pltpu.SemaphoreType.DMA,
        pltpu.SemaphoreType.DMA,
    ],
)
def _sc_edges(pa, pb, bw_mat, gate, row, col, out1, out2,
              idxr, idxc, wbuf, gbuf, ubuf, gubuf, sidx, S1, S2,
              semi0, semi1, semw0, semw1, semg0, semg1, sems):
    cid = lax.axis_index("c")
    sid = lax.axis_index("s")
    semi = (semi0, semi1)
    semw = (semw0, semw1)
    semg = (semg0, semg1)

    zv = jnp.zeros((L,), jnp.float32)

    def _zrow(i, carry):
        for j in range(H // L):
            ubuf[i, pl.ds(j * L, L)] = zv
        gubuf[i, pl.ds(0, L)] = zv
        return carry

    lax.fori_loop(0, B, _zrow, 0)
    nfull = RPT // B
    tail = RPT - nfull * B
    for j in range(nfull):
        pltpu.sync_copy(ubuf, S1.at[pl.ds(sid * RPT + j * B, B)])
        pltpu.sync_copy(gubuf, S2.at[pl.ds(sid * RPT + j * B, B)])
    pltpu.sync_copy(ubuf.at[pl.ds(0, tail)],
                    S1.at[pl.ds(sid * RPT + nfull * B, tail)])
    pltpu.sync_copy(gubuf.at[pl.ds(0, tail)],
                    S2.at[pl.ds(sid * RPT + nfull * B, tail)])
    plsc.subcore_barrier()

    ebase = (cid * NS + sid) * E_PER_TILE

    def _issue_idx(blk, b):
        off = ebase + blk * B
        pltpu.async_copy(row.at[pl.ds(off, B)], idxr.at[b], semi[b])
        pltpu.async_copy(col.at[pl.ds(off, B)], idxc.at[b], semi[b])

    def _wait_idx(b):
        pltpu.make_async_copy(row.at[pl.ds(0, B)], idxr.at[b], semi[b]).wait()
        pltpu.make_async_copy(col.at[pl.ds(0, B)], idxc.at[b], semi[b]).wait()

    def _issue_w(blk, b):
        off = ebase + blk * B
        pltpu.async_copy(bw_mat.at[pl.ds(off, B)], wbuf.at[b], semw[b])
        pltpu.async_copy(gate.at[pl.ds(off, B)], gbuf.at[b], semw[b])

    def _wait_w(b):
        pltpu.make_async_copy(bw_mat.at[pl.ds(0, B)], wbuf.at[b],
                              semw[b]).wait()
        pltpu.make_async_copy(gate.at[pl.ds(0, B)], gbuf.at[b], semw[b]).wait()

    def _issue_adds(b):
        pltpu.async_copy(pa.at[idxr.at[b]], wbuf.at[b], semg[b], add=True)
        pltpu.async_copy(pb.at[idxc.at[b]], wbuf.at[b], semg[b], add=True)

    def _wait_adds(b):
        pltpu.make_async_copy(pa.at[idxr.at[b]], wbuf.at[b], semg[b]).wait()
        pltpu.make_async_copy(pb.at[idxc.at[b]], wbuf.at[b], semg[b]).wait()

    def _issue_scat(b):
        pltpu.async_copy(ubuf, S1.at[sidx.at[b]], sems, add=True)
        pltpu.async_copy(gubuf, S2.at[sidx.at[b]], sems, add=True)

    def _wait_scat(b):
        pltpu.make_async_copy(ubuf, S1.at[sidx.at[b]], sems).wait()
        pltpu.make_async_copy(gubuf, S2.at[sidx.at[b]], sems).wait()

    def _snap_idx(b):
        for o in range(0, B - L + 1, L):
            sidx[b, pl.ds(o, L)] = idxr[b, pl.ds(o, L)]
        if B % L:
            sidx[b, pl.ds(B - L, L)] = idxr[b, pl.ds(B - L, L)]

    def _compute(b):
        def _edge(e, c2):
            gv = plsc.load_gather(gbuf.at[b],
                                  [lax.broadcast_in_dim(e, (L,), ())])
            for j in range(H // L):
                x = wbuf[b, e, pl.ds(j * L, L)]
                ubuf[e, pl.ds(j * L, L)] = jnp.maximum(x, 0.0) * gv
            gubuf[e, pl.ds(0, L)] = gv
            return c2

        lax.fori_loop(0, B, _edge, 0)

    def _half(j, b, wait_scat_prev, prefetch, adds_next):
        _wait_adds(b)
        _snap_idx(b)
        if prefetch:
            _issue_idx(j + 2, b)
        if adds_next:
            _wait_w(1 - b)
            _issue_adds(1 - b)
        if wait_scat_prev:
            _wait_scat(b)
        _compute(b)
        if prefetch:
            _wait_idx(b)
            _issue_w(j + 2, b)
        _issue_scat(b)

    _issue_idx(0, 0)
    _issue_idx(1, 1)
    _issue_w(0, 0)
    _issue_w(1, 1)
    _wait_idx(0)
    _wait_idx(1)
    _wait_w(0)
    _issue_adds(0)

    _half(0, 0, False, True, True)
    _half(1, 1, True, True, True)

    def _pair(k, carry):
        j = 2 * k
        _half(j, 0, True, True, True)
        _half(j + 1, 1, True, True, True)
        return carry

    lax.fori_loop(1, (NBLK - 3) // 2, _pair, 0)

    _half(NBLK - 3, 0, True, True, True)
    _half(NBLK - 2, 1, True, False, True)
    _half(NBLK - 1, 0, True, False, False)
    _wait_scat(0)

    plsc.subcore_barrier()
    pltpu.sync_copy(S1.at[pl.ds(sid * RPT, RPT)],
                    out1.at[cid, pl.ds(sid * RPT, RPT)])
    pltpu.sync_copy(S2.at[pl.ds(sid * RPT, RPT)],
                    out2.at[cid, pl.ds(sid * RPT, RPT)])


NB = 1000
EB = 3200


def _node_proj_body(h_ref, wa_ref, wb_ref, oa_ref, ob_ref):
    hh = h_ref[...]
    oa_ref[...] = jnp.dot(hh, wa_ref[...], preferred_element_type=jnp.float32)
    ob_ref[...] = jnp.dot(hh, wb_ref[...], preferred_element_type=jnp.float32)


def _node_proj(h, W1a, W1b):
    return pl.pallas_call(
        _node_proj_body,
        grid=(N_NODES // NB,),
        in_specs=[pl.BlockSpec((NB, H), lambda i: (i, 0)),
                  pl.BlockSpec((H, H), lambda i: (0, 0)),
                  pl.BlockSpec((H, H), lambda i: (0, 0))],
        out_specs=[pl.BlockSpec((NB, H), lambda i: (i, 0)),
                   pl.BlockSpec((NB, H), lambda i: (i, 0))],
        out_shape=[jax.ShapeDtypeStruct((N_NODES, H), jnp.float32),
                   jax.ShapeDtypeStruct((N_NODES, H), jnp.float32)],
    )(h, W1a, W1b)


def _edge_proj_body(bond_ref, w1c_ref, b1_ref, wa_ref, ba_ref, ow_ref, og_ref):
    bond = bond_ref[...]
    ow_ref[...] = (jnp.dot(bond, w1c_ref[...],
                           preferred_element_type=jnp.float32) + b1_ref[...])
    z = lax.dot_general(wa_ref[...], bond, (((1,), (1,)), ((), ())),
                        preferred_element_type=jnp.float32) + ba_ref[...]
    og_ref[...] = jax.nn.sigmoid(z)


def _edge_proj(bond_emb, W1c, b1, Wa, ba):
    bondw, gate_row = pl.pallas_call(
        _edge_proj_body,
        grid=(E // EB,),
        in_specs=[pl.BlockSpec((EB, BOND), lambda i: (i, 0)),
                  pl.BlockSpec((BOND, H), lambda i: (0, 0)),
                  pl.BlockSpec((1, H), lambda i: (0, 0)),
                  pl.BlockSpec((1, BOND), lambda i: (0, 0)),
                  pl.BlockSpec((1, 1), lambda i: (0, 0))],
        out_specs=[pl.BlockSpec((EB, H), lambda i: (i, 0)),
                   pl.BlockSpec((1, EB), lambda i: (0, i))],
        out_shape=[jax.ShapeDtypeStruct((E, H), jnp.float32),
                   jax.ShapeDtypeStruct((1, E), jnp.float32)],
    )(bond_emb, W1c, b1.reshape(1, H), Wa.reshape(1, BOND),
      ba.reshape(1, 1))
    return bondw, gate_row


def _epilogue_body(h_ref, s1_ref, s2_ref, w2_ref, b2_ref, o_ref):
    t = s1_ref[0] + s1_ref[1]
    cnt = (s2_ref[0] + s2_ref[1])[:, :1]
    o_ref[...] = (h_ref[...]
                  + jnp.dot(t, w2_ref[...], preferred_element_type=jnp.float32)
                  + cnt * b2_ref[...])


def _epilogue(h, s1_parts, s2_parts, W2, b2):
    return pl.pallas_call(
        _epilogue_body,
        grid=(N_NODES // NB,),
        in_specs=[pl.BlockSpec((NB, H), lambda i: (i, 0)),
                  pl.BlockSpec((NC, NB, H), lambda i: (0, i, 0)),
                  pl.BlockSpec((NC, NB, L), lambda i: (0, i, 0)),
                  pl.BlockSpec((H, H), lambda i: (0, 0)),
                  pl.BlockSpec((1, H), lambda i: (0, 0))],
        out_specs=pl.BlockSpec((NB, H), lambda i: (i, 0)),
        out_shape=jax.ShapeDtypeStruct((N_NODES, H), jnp.float32),
    )(h, s1_parts, s2_parts, W2, b2.reshape(1, H))


def kernel(h, edge_index, bond_emb, W1, b1, W2, b2, Wa, ba):
    row = edge_index[0].astype(jnp.int32)
    col = edge_index[1].astype(jnp.int32)
    W1a = W1[:H]
    W1b = W1[H:2 * H]
    W1c = W1[2 * H:]
    pa, pb = _node_proj(h, W1a, W1b)
    bondw, gate_row = _edge_proj(bond_emb, W1c, b1, Wa, ba)
    gate = gate_row.reshape(E)
    s1_parts, s2_parts = _sc_edges(pa, pb, bondw, gate, row, col)
    return _epilogue(h, s1_parts, s2_parts, W2, b2)


# speedup vs baseline: 1.7970x; 1.0115x over previous
import functools

import jax
import jax.numpy as jnp
from jax import lax
from jax.experimental import pallas as pl
from jax.experimental.pallas import tpu as pltpu
from jax.experimental.pallas import tpu_sc as plsc

H = 128
BOND = 64
N_NODES = 10000
E = 320000
L = 16
HA = H + L

NC = 2
NS = 16
E_PER_SC = E // NC
E_PER_TILE = E_PER_SC // NS
B = 80
NBLK = E_PER_TILE // B
RPT = N_NODES // NS

_mesh = plsc.VectorSubcoreMesh(core_axis_name="c", subcore_axis_name="s")


@functools.partial(
    pl.kernel,
    out_type=(jax.ShapeDtypeStruct((NC, N_NODES, H), jnp.float32),
              jax.ShapeDtypeStruct((NC, N_NODES, L), jnp.float32)),
    mesh=_mesh,
    compiler_params=pltpu.CompilerParams(use_tc_tiling_on_sc=False,
                                         needs_layout_passes=False),
    scratch_types=[
        pltpu.VMEM((2, 2, B), jnp.int32),
        pltpu.VMEM((2, B, H), jnp.float32),
        pltpu.VMEM((2, B), jnp.float32),
        pltpu.VMEM((B, H), jnp.float32),
        pltpu.VMEM((B, L), jnp.float32),
        pltpu.VMEM((2, B), jnp.int32),
        pltpu.VMEM_SHARED((N_NODES, H), jnp.float32),
        pltpu.VMEM_SHARED((N_NODES, L), jnp.float32),
        pltpu.SemaphoreType.DMA,
        pltpu.SemaphoreType.DMA,
        pltpu.SemaphoreType.DMA,
        pltpu.SemaphoreType.DMA,
        pltpu.SemaphoreType.DMA,
        pltpu.SemaphoreType.DMA,
        pltpu.SemaphoreType.DMA,
    ],
)
def _sc_edges(pa, pb, bw_mat, gate, eidx, out1, out2,
              idxrc, wbuf, gbuf, ubuf, gubuf, sidx, S1, S2,
              semi0, semi1, semw0, semw1, semg0, semg1, sems):
    cid = lax.axis_index("c")
    sid = lax.axis_index("s")
    semi = (semi0, semi1)
    semw = (semw0, semw1)
    semg = (semg0, semg1)

    zv = jnp.zeros((L,), jnp.float32)

    def _zrow(i, carry):
        for j in range(H // L):
            ubuf[i, pl.ds(j * L, L)] = zv
        gubuf[i, pl.ds(0, L)] = zv
        return carry

    lax.fori_loop(0, B, _zrow, 0)
    nfull = RPT // B
    tail = RPT - nfull * B
    for j in range(nfull):
        pltpu.sync_copy(ubuf, S1.at[pl.ds(sid * RPT + j * B, B)])
        pltpu.sync_copy(gubuf, S2.at[pl.ds(sid * RPT + j * B, B)])
    pltpu.sync_copy(ubuf.at[pl.ds(0, tail)],
                    S1.at[pl.ds(sid * RPT + nfull * B, tail)])
    pltpu.sync_copy(gubuf.at[pl.ds(0, tail)],
                    S2.at[pl.ds(sid * RPT + nfull * B, tail)])
    plsc.subcore_barrier()

    ebase = (cid * NS + sid) * E_PER_TILE

    def _issue_idx(blk, b):
        off = ebase + blk * B
        pltpu.async_copy(eidx.at[:, pl.ds(off, B)], idxrc.at[b], semi[b])

    def _wait_idx(b):
        pltpu.make_async_copy(eidx.at[:, pl.ds(0, B)], idxrc.at[b],
                              semi[b]).wait()

    def _issue_w(blk, b):
        off = ebase + blk * B
        pltpu.async_copy(bw_mat.at[pl.ds(off, B)], wbuf.at[b], semw[b])
        pltpu.async_copy(gate.at[pl.ds(off, B)], gbuf.at[b], semw[b])

    def _wait_w(b):
        pltpu.make_async_copy(bw_mat.at[pl.ds(0, B)], wbuf.at[b],
                              semw[b]).wait()
        pltpu.make_async_copy(gate.at[pl.ds(0, B)], gbuf.at[b], semw[b]).wait()

    def _issue_adds(b):
        pltpu.async_copy(pa.at[idxrc.at[b, 0]], wbuf.at[b], semg[b], add=True)
        pltpu.async_copy(pb.at[idxrc.at[b, 1]], wbuf.at[b], semg[b], add=True)

    def _wait_adds(b):
        pltpu.make_async_copy(pa.at[idxrc.at[b, 0]], wbuf.at[b],
                              semg[b]).wait()
        pltpu.make_async_copy(pb.at[idxrc.at[b, 1]], wbuf.at[b],
                              semg[b]).wait()

    def _issue_scat(b):
        pltpu.async_copy(ubuf, S1.at[sidx.at[b]], sems, add=True)
        pltpu.async_copy(gubuf, S2.at[sidx.at[b]], sems, add=True)

    def _wait_scat(b):
        pltpu.make_async_copy(ubuf, S1.at[sidx.at[b]], sems).wait()
        pltpu.make_async_copy(gubuf, S2.at[sidx.at[b]], sems).wait()

    def _snap_idx(b):
        for o in range(0, B - L + 1, L):
            sidx[b, pl.ds(o, L)] = idxrc[b, 0, pl.ds(o, L)]
        if B % L:
            sidx[b, pl.ds(B - L, L)] = idxrc[b, 0, pl.ds(B - L, L)]

    def _compute(b):
        def _edge(e, c2):
            gv = plsc.load_gather(gbuf.at[b],
                                  [lax.broadcast_in_dim(e, (L,), ())])
            for j in range(H // L):
                x = wbuf[b, e, pl.ds(j * L, L)]
                ubuf[e, pl.ds(j * L, L)] = jnp.maximum(x, 0.0) * gv
            gubuf[e, pl.ds(0, L)] = gv
            return c2

        lax.fori_loop(0, B, _edge, 0)

    def _half(j, b, wait_scat_prev, prefetch, adds_next):
        _wait_adds(b)
        _snap_idx(b)
        if prefetch:
            _issue_idx(j + 2, b)
        if adds_next:
            _wait_w(1 - b)
            _issue_adds(1 - b)
        if wait_scat_prev:
            _wait_scat(b)
        _compute(b)
        if prefetch:
            _wait_idx(b)
            _issue_w(j + 2, b)
        _issue_scat(b)

    _issue_idx(0, 0)
    _issue_idx(1, 1)
    _issue_w(0, 0)
    _issue_w(1, 1)
    _wait_idx(0)
    _wait_idx(1)
    _wait_w(0)
    _issue_adds(0)

    _half(0, 0, False, True, True)
    _half(1, 1, True, True, True)

    def _pair(k, carry):
        j = 2 * k
        _half(j, 0, True, True, True)
        _half(j + 1, 1, True, True, True)
        return carry

    lax.fori_loop(1, (NBLK - 3) // 2, _pair, 0)

    _half(NBLK - 3, 0, True, True, True)
    _half(NBLK - 2, 1, True, False, True)
    _half(NBLK - 1, 0, True, False, False)
    _wait_scat(0)

    plsc.subcore_barrier()
    pltpu.sync_copy(S1.at[pl.ds(sid * RPT, RPT)],
                    out1.at[cid, pl.ds(sid * RPT, RPT)])
    pltpu.sync_copy(S2.at[pl.ds(sid * RPT, RPT)],
                    out2.at[cid, pl.ds(sid * RPT, RPT)])


NB = 1000
EB = 3200


def _node_proj_body(h_ref, wa_ref, wb_ref, oa_ref, ob_ref):
    hh = h_ref[...]
    oa_ref[...] = jnp.dot(hh, wa_ref[...], preferred_element_type=jnp.float32)
    ob_ref[...] = jnp.dot(hh, wb_ref[...], preferred_element_type=jnp.float32)


def _node_proj(h, W1a, W1b):
    return pl.pallas_call(
        _node_proj_body,
        grid=(N_NODES // NB,),
        in_specs=[pl.BlockSpec((NB, H), lambda i: (i, 0)),
                  pl.BlockSpec((H, H), lambda i: (0, 0)),
                  pl.BlockSpec((H, H), lambda i: (0, 0))],
        out_specs=[pl.BlockSpec((NB, H), lambda i: (i, 0)),
                   pl.BlockSpec((NB, H), lambda i: (i, 0))],
        out_shape=[jax.ShapeDtypeStruct((N_NODES, H), jnp.float32),
                   jax.ShapeDtypeStruct((N_NODES, H), jnp.float32)],
    )(h, W1a, W1b)


def _edge_proj_body(bond_ref, w1c_ref, b1_ref, wa_ref, ba_ref, ow_ref, og_ref):
    bond = bond_ref[...]
    ow_ref[...] = (jnp.dot(bond, w1c_ref[...],
                           preferred_element_type=jnp.float32) + b1_ref[...])
    z = lax.dot_general(wa_ref[...], bond, (((1,), (1,)), ((), ())),
                        preferred_element_type=jnp.float32) + ba_ref[...]
    og_ref[...] = jax.nn.sigmoid(z)


def _edge_proj(bond_emb, W1c, b1, Wa, ba):
    bondw, gate_row = pl.pallas_call(
        _edge_proj_body,
        grid=(E // EB,),
        in_specs=[pl.BlockSpec((EB, BOND), lambda i: (i, 0)),
                  pl.BlockSpec((BOND, H), lambda i: (0, 0)),
                  pl.BlockSpec((1, H), lambda i: (0, 0)),
                  pl.BlockSpec((1, BOND), lambda i: (0, 0)),
                  pl.BlockSpec((1, 1), lambda i: (0, 0))],
        out_specs=[pl.BlockSpec((EB, H), lambda i: (i, 0)),
                   pl.BlockSpec((1, EB), lambda i: (0, i))],
        out_shape=[jax.ShapeDtypeStruct((E, H), jnp.float32),
                   jax.ShapeDtypeStruct((1, E), jnp.float32)],
    )(bond_emb, W1c, b1.reshape(1, H), Wa.reshape(1, BOND),
      ba.reshape(1, 1))
    return bondw, gate_row


def _epilogue_body(h_ref, s1_ref, s2_ref, w2_ref, b2_ref, o_ref):
    t = s1_ref[0] + s1_ref[1]
    cnt = (s2_ref[0] + s2_ref[1])[:, :1]
    o_ref[...] = (h_ref[...]
                  + jnp.dot(t, w2_ref[...], preferred_element_type=jnp.float32)
                  + cnt * b2_ref[...])


def _epilogue(h, s1_parts, s2_parts, W2, b2):
    return pl.pallas_call(
        _epilogue_body,
        grid=(N_NODES // NB,),
        in_specs=[pl.BlockSpec((NB, H), lambda i: (i, 0)),
                  pl.BlockSpec((NC, NB, H), lambda i: (0, i, 0)),
                  pl.BlockSpec((NC, NB, L), lambda i: (0, i, 0)),
                  pl.BlockSpec((H, H), lambda i: (0, 0)),
                  pl.BlockSpec((1, H), lambda i: (0, 0))],
        out_specs=pl.BlockSpec((NB, H), lambda i: (i, 0)),
        out_shape=jax.ShapeDtypeStruct((N_NODES, H), jnp.float32),
    )(h, s1_parts, s2_parts, W2, b2.reshape(1, H))


def kernel(h, edge_index, bond_emb, W1, b1, W2, b2, Wa, ba):
    eidx = edge_index.astype(jnp.int32)
    W1a = W1[:H]
    W1b = W1[H:2 * H]
    W1c = W1[2 * H:]
    pa, pb = _node_proj(h, W1a, W1b)
    bondw, gate_row = _edge_proj(bond_emb, W1c, b1, Wa, ba)
    gate = gate_row.reshape(E)
    s1_parts, s2_parts = _sc_edges(pa, pb, bondw, gate, eidx)
    return _epilogue(h, s1_parts, s2_parts, W2, b2)
